# baseline (speedup 1.0000x reference)
import functools

import jax
import jax.numpy as jnp
from jax import lax
from jax.experimental import pallas as pl
from jax.experimental.pallas import tpu as pltpu
from jax.experimental.pallas import tpu_sc as plsc

F = 128
K = 16
N = 10000
E = 320000

_NC = 2
_NS = 16
_NW = _NC * _NS
_EPW = E // _NW
_C = 80
_NCH = _EPW // _C
_NP = 10240
_RPS = _NP // _NS

_NB = 2000


def _dot_t(x, w):
    return lax.dot_general(x, w, (((1,), (1,)), ((), ())),
                           preferred_element_type=jnp.float32)


_GSCALE = 8192.0


def _gate_body(rbf_ref, wg_ref, x_ref, wd_ref, bd_ref, ei_ref, gate_ref,
               y_ref, pidx_ref):
    res = _dot_t(rbf_ref[...], wg_ref[...])
    q = res * _GSCALE
    q = jnp.clip(q + jnp.where(q >= 0, 0.5, -0.5), -32767.0, 32767.0)
    qi = q.astype(jnp.int32).reshape(_EPW // 2, 2, F)
    gate_ref[...] = (qi[:, 0, :] & jnp.int32(0xFFFF)) | (qi[:, 1, :] << 16)

    @pl.when(pl.program_id(0) == 0)
    def _():
        xa = jnp.maximum(x_ref[...], 0.0)
        y = jnp.maximum(_dot_t(xa, wd_ref[...]) + bd_ref[...], 0.0)
        y_ref[...] = y * (1.0 / _GSCALE)
        ei = ei_ref[...]
        pidx_ref[...] = ei[0] | (ei[1] << 16)


def _gate_call(rbf, w_g, x, w_diff, b_diff, edge_index):
    return pl.pallas_call(
        _gate_body,
        grid=(_NW,),
        in_specs=[
            pl.BlockSpec((_EPW, K), lambda i: (i, 0)),
            pl.BlockSpec((F, K), lambda i: (0, 0)),
            pl.BlockSpec((N, F), lambda i: (0, 0)),
            pl.BlockSpec((F, F), lambda i: (0, 0)),
            pl.BlockSpec((1, F), lambda i: (0, 0)),
            pl.BlockSpec((2, E), lambda i: (0, 0)),
        ],
        out_specs=[
            pl.BlockSpec((_EPW // 2, F), lambda i: (i, 0)),
            pl.BlockSpec((N, F), lambda i: (0, 0)),
            pl.BlockSpec((E,), lambda i: (0,)),
        ],
        out_shape=[
            jax.ShapeDtypeStruct((E // 2, F), jnp.int32),
            jax.ShapeDtypeStruct((N, F), jnp.float32),
            jax.ShapeDtypeStruct((E,), jnp.int32),
        ],
    )(rbf, w_g, x, w_diff, b_diff, edge_index)


def _sc_body(y_hbm, gate_hbm, pidx_hbm, out_hbm,
             pidx0, pidx1, sidx0, sidx1, didx0, didx1,
             ybuf0, ybuf1, gbuf0, gbuf1, aggr,
             gsem0, gsem1, lsem0, lsem1, ssem0, ssem1,
             isem0, isem1):
    cid = lax.axis_index("c")
    sid = lax.axis_index("s")
    wid = cid * _NS + sid
    ebase = wid * _EPW

    pidxs = (pidx0, pidx1)
    sidxs = (sidx0, sidx1)
    didxs = (didx0, didx1)
    ybufs = (ybuf0, ybuf1)
    gbufs = (gbuf0, gbuf1)
    gsems = (gsem0, gsem1)
    lsems = (lsem0, lsem1)
    ssems = (ssem0, ssem1)
    isems = (isem0, isem1)

    zero = jnp.zeros((16,), jnp.float32)

    def zrow(r, carry):
        for c8 in range(F // 16):
            ybuf0[r, pl.ds(c8 * 16, 16)] = zero
        return carry

    lax.fori_loop(0, _C, zrow, 0)
    for k in range(_RPS // _C):
        pltpu.sync_copy(ybuf0, aggr.at[pl.ds(sid * _RPS + k * _C, _C)])

    def idx_copy(i, b):
        pltpu.async_copy(pidx_hbm.at[pl.ds(ebase + i * _C, _C)], pidxs[b],
                         isems[b])

    def idx_extract(i, b):
        pltpu.make_async_copy(pidx_hbm.at[pl.ds(ebase + i * _C, _C)],
                              pidxs[b], isems[b]).wait()
        for k in range(_C // 16):
            s = pl.ds(k * 16, 16)
            v = pidxs[b][s]
            sidxs[b][s] = v & jnp.int32(0xFFFF)
            didxs[b][s] = v >> 16

    gbase = wid * (_EPW // 2)

    def fetch(i, b):
        pltpu.async_copy(y_hbm.at[sidxs[b]], ybufs[b], gsems[b])
        goff = pl.multiple_of(gbase + i * (_C // 2), 8)
        pltpu.async_copy(gate_hbm.at[pl.ds(goff, _C // 2)], gbufs[b],
                         lsems[b])

    def fetch_wait(i, b):
        pltpu.make_async_copy(y_hbm.at[sidxs[b]], ybufs[b], gsems[b]).wait()
        goff = pl.multiple_of(gbase + i * (_C // 2), 8)
        pltpu.make_async_copy(gate_hbm.at[pl.ds(goff, _C // 2)], gbufs[b],
                              lsems[b]).wait()

    def mul(b):
        @plsc.parallel_loop(0, _C // 2, step=1, unroll=2)
        def _mul(q):
            for g in range(F // 16):
                s = pl.ds(g * 16, 16)
                g32 = gbufs[b][q, s]
                ga = ((g32 << 16) >> 16).astype(jnp.float32)
                gb = (g32 >> 16).astype(jnp.float32)
                ybufs[b][2 * q, s] = ybufs[b][2 * q, s] * ga
                ybufs[b][2 * q + 1, s] = ybufs[b][2 * q + 1, s] * gb

    def process(i, b, sw, pf, pc):
        fetch_wait(i, b)
        nb = 1 - b
        if sw:
            pltpu.make_async_copy(ybufs[nb], aggr.at[didxs[nb]],
                                  ssems[nb]).wait()
        if pf:
            idx_extract(i + 1, nb)
            fetch(i + 1, nb)
        mul(b)
        if pc:
            idx_copy(i + 2, b)
        pltpu.async_copy(ybufs[b], aggr.at[didxs[b]], ssems[b], add=True)

    idx_copy(0, 0)
    idx_copy(1, 1)
    idx_extract(0, 0)
    fetch(0, 0)
    plsc.subcore_barrier()
    process(0, 0, sw=False, pf=True, pc=True)
    process(1, 1, sw=True, pf=True, pc=True)

    def pair(j, carry):
        process(2 + 2 * j, 0, sw=True, pf=True, pc=True)
        process(3 + 2 * j, 1, sw=True, pf=True, pc=True)
        return carry

    lax.fori_loop(0, (_NCH - 5) // 2, pair, 0)

    process(_NCH - 3, 0, sw=True, pf=True, pc=True)
    process(_NCH - 2, 1, sw=True, pf=True, pc=False)
    process(_NCH - 1, 0, sw=True, pf=False, pc=False)

    pltpu.make_async_copy(ybuf0, aggr.at[didx0], ssem0).wait()
    plsc.subcore_barrier()

    pltpu.sync_copy(aggr.at[pl.ds(sid * _RPS, _RPS)],
                    out_hbm.at[cid, pl.ds(sid * _RPS, _RPS)])


@functools.lru_cache(maxsize=1)
def _sc_aggregate():
    mesh = plsc.VectorSubcoreMesh(core_axis_name="c", subcore_axis_name="s",
                                  num_cores=_NC, num_subcores=_NS)
    return pl.kernel(
        _sc_body,
        out_type=jax.ShapeDtypeStruct((_NC, _NP, F), jnp.float32),
        mesh=mesh,
        scratch_types=[
            pltpu.VMEM((_C,), jnp.int32),
            pltpu.VMEM((_C,), jnp.int32),
            pltpu.VMEM((_C,), jnp.int32),
            pltpu.VMEM((_C,), jnp.int32),
            pltpu.VMEM((_C,), jnp.int32),
            pltpu.VMEM((_C,), jnp.int32),
            pltpu.VMEM((_C, F), jnp.float32),
            pltpu.VMEM((_C, F), jnp.float32),
            pltpu.VMEM((_C // 2, F), jnp.int32),
            pltpu.VMEM((_C // 2, F), jnp.int32),
            pltpu.VMEM_SHARED((_NP, F), jnp.float32),
            pltpu.SemaphoreType.DMA,
            pltpu.SemaphoreType.DMA,
            pltpu.SemaphoreType.DMA,
            pltpu.SemaphoreType.DMA,
            pltpu.SemaphoreType.DMA,
            pltpu.SemaphoreType.DMA,
            pltpu.SemaphoreType.DMA,
            pltpu.SemaphoreType.DMA,
        ],
    )


def _post_body(p_ref, x_ref, u_ref, ws_ref, bs_ref, wr1_ref, br1_ref,
               wr2_ref, br2_ref, wl_ref, bl_ref, out_ref, mx_ref):
    p = p_ref[...]
    xa = jnp.maximum(x_ref[...], 0.0)
    a = jnp.maximum(_dot_t(xa, ws_ref[...]) + bs_ref[...], 0.0)
    mx = a + p[0] + p[1]
    mx_ref[...] = mx
    t = jnp.maximum(mx, 0.0)
    t = jnp.maximum(_dot_t(t, wr1_ref[...]) + br1_ref[...], 0.0)
    t = _dot_t(t, wr2_ref[...]) + br2_ref[...]
    h = mx + t
    v = jnp.maximum(h, 0.0)
    v = _dot_t(v, wl_ref[...]) + bl_ref[...]
    out_ref[...] = v + x_ref[...] * u_ref[...]


def _post_call(partials, x, u, w_same, b_same, w_r1, b_r1, w_r2, b_r2,
               w_last, b_last):
    grid = N // _NB
    return pl.pallas_call(
        _post_body,
        grid=(grid,),
        in_specs=[
            pl.BlockSpec((_NC, _NB, F), lambda i: (0, i, 0)),
            pl.BlockSpec((_NB, F), lambda i: (i, 0)),
            pl.BlockSpec((1, F), lambda i: (0, 0)),
            pl.BlockSpec((F, F), lambda i: (0, 0)),
            pl.BlockSpec((1, F), lambda i: (0, 0)),
            pl.BlockSpec((F, F), lambda i: (0, 0)),
            pl.BlockSpec((1, F), lambda i: (0, 0)),
            pl.BlockSpec((F, F), lambda i: (0, 0)),
            pl.BlockSpec((1, F), lambda i: (0, 0)),
            pl.BlockSpec((F, F), lambda i: (0, 0)),
            pl.BlockSpec((1, F), lambda i: (0, 0)),
        ],
        out_specs=[
            pl.BlockSpec((_NB, F), lambda i: (i, 0)),
            pl.BlockSpec((_NB, F), lambda i: (i, 0)),
        ],
        out_shape=[
            jax.ShapeDtypeStruct((N, F), jnp.float32),
            jax.ShapeDtypeStruct((N, F), jnp.float32),
        ],
    )(partials, x, u, w_same, b_same, w_r1, b_r1, w_r2, b_r2, w_last, b_last)


def kernel(x, edge_index, rbf, W_same, b_same, W_diff, b_diff, W_G, u,
           W_r1, b_r1, W_r2, b_r2, W_last, b_last):
    gate, y, pidx = _gate_call(rbf, W_G, x, W_diff, b_diff.reshape(1, F),
                               edge_index)
    partials = _sc_aggregate()(y, gate, pidx)
    out, msged_x = _post_call(partials, x, u, W_same, b_same.reshape(1, F),
                              W_r1, b_r1.reshape(1, F), W_r2,
                              b_r2.reshape(1, F), W_last, b_last.reshape(1, F))
    return (out, msged_x)


# speedup vs baseline: 1.2842x; 1.2842x over previous
import functools

import jax
import jax.numpy as jnp
from jax import lax
from jax.experimental import pallas as pl
from jax.experimental.pallas import tpu as pltpu
from jax.experimental.pallas import tpu_sc as plsc

F = 128
K = 16
N = 10000
E = 320000

_NC = 2
_NS = 16
_NW = _NC * _NS
_EPW = E // _NW
_C = 80
_NCH = _EPW // _C
_NP = 10240
_RPS = _NP // _NS

_NB = 2000


def _dot_t(x, w):
    return lax.dot_general(x, w, (((1,), (1,)), ((), ())),
                           preferred_element_type=jnp.float32)


_GSCALE = 8192.0


def _gate_body(rbf_ref, wg_ref, x_ref, wd_ref, bd_ref, ei_ref, gate_ref,
               y_ref, pidx_ref):
    res = _dot_t(rbf_ref[...], wg_ref[...])
    q = res * _GSCALE
    q = jnp.clip(q + jnp.where(q >= 0, 0.5, -0.5), -32767.0, 32767.0)
    qi = q.astype(jnp.int32)
    h = _EPW // 2
    gate_ref[...] = (qi[:h] & jnp.int32(0xFFFF)) | (qi[h:] << 16)

    @pl.when(pl.program_id(0) == 0)
    def _():
        xa = jnp.maximum(x_ref[...], 0.0)
        y = jnp.maximum(_dot_t(xa, wd_ref[...]) + bd_ref[...], 0.0)
        y_ref[...] = y * (1.0 / _GSCALE)
        ei = ei_ref[...]
        pidx_ref[...] = ei[0] | (ei[1] << 16)


def _gate_call(rbf, w_g, x, w_diff, b_diff, edge_index):
    return pl.pallas_call(
        _gate_body,
        grid=(_NW,),
        in_specs=[
            pl.BlockSpec((_EPW, K), lambda i: (i, 0)),
            pl.BlockSpec((F, K), lambda i: (0, 0)),
            pl.BlockSpec((N, F), lambda i: (0, 0)),
            pl.BlockSpec((F, F), lambda i: (0, 0)),
            pl.BlockSpec((1, F), lambda i: (0, 0)),
            pl.BlockSpec((2, E), lambda i: (0, 0)),
        ],
        out_specs=[
            pl.BlockSpec((_EPW // 2, F), lambda i: (i, 0)),
            pl.BlockSpec((N, F), lambda i: (0, 0)),
            pl.BlockSpec((E,), lambda i: (0,)),
        ],
        out_shape=[
            jax.ShapeDtypeStruct((E // 2, F), jnp.int32),
            jax.ShapeDtypeStruct((N, F), jnp.float32),
            jax.ShapeDtypeStruct((E,), jnp.int32),
        ],
    )(rbf, w_g, x, w_diff, b_diff, edge_index)


def _sc_body(y_hbm, gate_hbm, pidx_hbm, out_hbm,
             pidx0, pidx1, sidx0, sidx1, didx0, didx1,
             ybuf0, ybuf1, gbuf0, gbuf1, aggr,
             gsem0, gsem1, lsem0, lsem1, ssem0, ssem1,
             isem0, isem1):
    cid = lax.axis_index("c")
    sid = lax.axis_index("s")
    wid = cid * _NS + sid
    ebase = wid * _EPW

    pidxs = (pidx0, pidx1)
    sidxs = (sidx0, sidx1)
    didxs = (didx0, didx1)
    ybufs = (ybuf0, ybuf1)
    gbufs = (gbuf0, gbuf1)
    gsems = (gsem0, gsem1)
    lsems = (lsem0, lsem1)
    ssems = (ssem0, ssem1)
    isems = (isem0, isem1)

    zero = jnp.zeros((16,), jnp.float32)

    def zrow(r, carry):
        for c8 in range(F // 16):
            ybuf0[r, pl.ds(c8 * 16, 16)] = zero
        return carry

    lax.fori_loop(0, _C, zrow, 0)
    for k in range(_RPS // _C):
        pltpu.sync_copy(ybuf0, aggr.at[pl.ds(sid * _RPS + k * _C, _C)])

    _H = _C // 2

    def idx_copy(i, b):
        pltpu.async_copy(pidx_hbm.at[pl.ds(ebase + i * _H, _H)],
                         pidxs[b].at[pl.ds(0, _H)], isems[b])
        pltpu.async_copy(pidx_hbm.at[pl.ds(ebase + _EPW // 2 + i * _H, _H)],
                         pidxs[b].at[pl.ds(_H, _H)], isems[b])

    def idx_extract(i, b):
        pltpu.make_async_copy(pidx_hbm.at[pl.ds(ebase + i * _H, _H)],
                              pidxs[b].at[pl.ds(0, _H)], isems[b]).wait()
        pltpu.make_async_copy(pidx_hbm.at[pl.ds(ebase + i * _H, _H)],
                              pidxs[b].at[pl.ds(_H, _H)], isems[b]).wait()
        for k in range(_C // 16):
            s = pl.ds(k * 16, 16)
            v = pidxs[b][s]
            sidxs[b][s] = v & jnp.int32(0xFFFF)
            didxs[b][s] = v >> 16

    gbase = wid * (_EPW // 2)

    def fetch(i, b):
        pltpu.async_copy(y_hbm.at[sidxs[b]], ybufs[b], gsems[b])
        goff = pl.multiple_of(gbase + i * (_C // 2), 8)
        pltpu.async_copy(gate_hbm.at[pl.ds(goff, _C // 2)], gbufs[b],
                         lsems[b])

    def fetch_wait(i, b):
        pltpu.make_async_copy(y_hbm.at[sidxs[b]], ybufs[b], gsems[b]).wait()
        goff = pl.multiple_of(gbase + i * (_C // 2), 8)
        pltpu.make_async_copy(gate_hbm.at[pl.ds(goff, _C // 2)], gbufs[b],
                              lsems[b]).wait()

    def mul(b):
        @plsc.parallel_loop(0, _C // 2, step=1, unroll=2)
        def _mul(q):
            for g in range(F // 16):
                s = pl.ds(g * 16, 16)
                g32 = gbufs[b][q, s]
                ga = ((g32 << 16) >> 16).astype(jnp.float32)
                gb = (g32 >> 16).astype(jnp.float32)
                ybufs[b][q, s] = ybufs[b][q, s] * ga
                qh = q + _C // 2
                ybufs[b][qh, s] = ybufs[b][qh, s] * gb

    def process(i, b, sw, pf, pc):
        fetch_wait(i, b)
        nb = 1 - b
        if sw:
            pltpu.make_async_copy(ybufs[nb], aggr.at[didxs[nb]],
                                  ssems[nb]).wait()
        if pf:
            idx_extract(i + 1, nb)
            fetch(i + 1, nb)
        mul(b)
        if pc:
            idx_copy(i + 2, b)
        pltpu.async_copy(ybufs[b], aggr.at[didxs[b]], ssems[b], add=True)

    idx_copy(0, 0)
    idx_copy(1, 1)
    idx_extract(0, 0)
    fetch(0, 0)
    plsc.subcore_barrier()
    process(0, 0, sw=False, pf=True, pc=True)
    process(1, 1, sw=True, pf=True, pc=True)

    def pair(j, carry):
        process(2 + 2 * j, 0, sw=True, pf=True, pc=True)
        process(3 + 2 * j, 1, sw=True, pf=True, pc=True)
        return carry

    lax.fori_loop(0, (_NCH - 5) // 2, pair, 0)

    process(_NCH - 3, 0, sw=True, pf=True, pc=True)
    process(_NCH - 2, 1, sw=True, pf=True, pc=False)
    process(_NCH - 1, 0, sw=True, pf=False, pc=False)

    pltpu.make_async_copy(ybuf0, aggr.at[didx0], ssem0).wait()
    plsc.subcore_barrier()

    pltpu.sync_copy(aggr.at[pl.ds(sid * _RPS, _RPS)],
                    out_hbm.at[cid, pl.ds(sid * _RPS, _RPS)])


@functools.lru_cache(maxsize=1)
def _sc_aggregate():
    mesh = plsc.VectorSubcoreMesh(core_axis_name="c", subcore_axis_name="s",
                                  num_cores=_NC, num_subcores=_NS)
    return pl.kernel(
        _sc_body,
        out_type=jax.ShapeDtypeStruct((_NC, _NP, F), jnp.float32),
        mesh=mesh,
        scratch_types=[
            pltpu.VMEM((_C,), jnp.int32),
            pltpu.VMEM((_C,), jnp.int32),
            pltpu.VMEM((_C,), jnp.int32),
            pltpu.VMEM((_C,), jnp.int32),
            pltpu.VMEM((_C,), jnp.int32),
            pltpu.VMEM((_C,), jnp.int32),
            pltpu.VMEM((_C, F), jnp.float32),
            pltpu.VMEM((_C, F), jnp.float32),
            pltpu.VMEM((_C // 2, F), jnp.int32),
            pltpu.VMEM((_C // 2, F), jnp.int32),
            pltpu.VMEM_SHARED((_NP, F), jnp.float32),
            pltpu.SemaphoreType.DMA,
            pltpu.SemaphoreType.DMA,
            pltpu.SemaphoreType.DMA,
            pltpu.SemaphoreType.DMA,
            pltpu.SemaphoreType.DMA,
            pltpu.SemaphoreType.DMA,
            pltpu.SemaphoreType.DMA,
            pltpu.SemaphoreType.DMA,
        ],
    )


def _post_body(p_ref, x_ref, u_ref, ws_ref, bs_ref, wr1_ref, br1_ref,
               wr2_ref, br2_ref, wl_ref, bl_ref, out_ref, mx_ref):
    p = p_ref[...]
    xa = jnp.maximum(x_ref[...], 0.0)
    a = jnp.maximum(_dot_t(xa, ws_ref[...]) + bs_ref[...], 0.0)
    mx = a + p[0] + p[1]
    mx_ref[...] = mx
    t = jnp.maximum(mx, 0.0)
    t = jnp.maximum(_dot_t(t, wr1_ref[...]) + br1_ref[...], 0.0)
    t = _dot_t(t, wr2_ref[...]) + br2_ref[...]
    h = mx + t
    v = jnp.maximum(h, 0.0)
    v = _dot_t(v, wl_ref[...]) + bl_ref[...]
    out_ref[...] = v + x_ref[...] * u_ref[...]


def _post_call(partials, x, u, w_same, b_same, w_r1, b_r1, w_r2, b_r2,
               w_last, b_last):
    grid = N // _NB
    return pl.pallas_call(
        _post_body,
        grid=(grid,),
        in_specs=[
            pl.BlockSpec((_NC, _NB, F), lambda i: (0, i, 0)),
            pl.BlockSpec((_NB, F), lambda i: (i, 0)),
            pl.BlockSpec((1, F), lambda i: (0, 0)),
            pl.BlockSpec((F, F), lambda i: (0, 0)),
            pl.BlockSpec((1, F), lambda i: (0, 0)),
            pl.BlockSpec((F, F), lambda i: (0, 0)),
            pl.BlockSpec((1, F), lambda i: (0, 0)),
            pl.BlockSpec((F, F), lambda i: (0, 0)),
            pl.BlockSpec((1, F), lambda i: (0, 0)),
            pl.BlockSpec((F, F), lambda i: (0, 0)),
            pl.BlockSpec((1, F), lambda i: (0, 0)),
        ],
        out_specs=[
            pl.BlockSpec((_NB, F), lambda i: (i, 0)),
            pl.BlockSpec((_NB, F), lambda i: (i, 0)),
        ],
        out_shape=[
            jax.ShapeDtypeStruct((N, F), jnp.float32),
            jax.ShapeDtypeStruct((N, F), jnp.float32),
        ],
    )(partials, x, u, w_same, b_same, w_r1, b_r1, w_r2, b_r2, w_last, b_last)


def kernel(x, edge_index, rbf, W_same, b_same, W_diff, b_diff, W_G, u,
           W_r1, b_r1, W_r2, b_r2, W_last, b_last):
    gate, y, pidx = _gate_call(rbf, W_G, x, W_diff, b_diff.reshape(1, F),
                               edge_index)
    partials = _sc_aggregate()(y, gate, pidx)
    out, msged_x = _post_call(partials, x, u, W_same, b_same.reshape(1, F),
                              W_r1, b_r1.reshape(1, F), W_r2,
                              b_r2.reshape(1, F), W_last, b_last.reshape(1, F))
    return (out, msged_x)
